# Initial kernel scaffold; baseline (speedup 1.0000x reference)
import jax
import jax.numpy as jnp
from jax.experimental import pallas as pl


def kernel(x, emb_tables, fc_table, bias):
    raise NotImplementedError("write your pallas kernel here")


# speedup vs baseline: 8.9157x; 8.9157x over previous
import functools

import jax
import jax.numpy as jnp
import numpy as np
from jax import lax
from jax.experimental import pallas as pl
from jax.experimental.pallas import tpu as pltpu
from jax.experimental.pallas import tpu_sc as plsc

F = 26
FP = 32
V = 100000
D = 16
B = 4096
NC = 2
NS = 16
NW = NC * NS
SPW = B // NW
P = F * F
PP = (P + 15) // 16 * 16
NJ = PP // 16


def _consts():
    p = np.arange(PP)
    s = np.where(p < P, p // F, 0)
    sel_a = np.minimum(s, 15).astype(np.int32).reshape(NJ, 16)
    sel_b = np.maximum(s - 16, 0).astype(np.int32).reshape(NJ, 16)
    in_a = (s < 16).astype(np.int32).reshape(NJ, 16)
    off = np.where(p < P, (p % F) * V, 0).astype(np.int32).reshape(NJ, 16)
    return (jnp.asarray(sel_a), jnp.asarray(sel_b), jnp.asarray(in_a),
            jnp.asarray(off))


def _take16(vec, idx):
    return vec.at[idx].get(mode="promise_in_bounds")


@functools.partial(
    pl.kernel,
    out_type=jax.ShapeDtypeStruct((B,), jnp.float32),
    mesh=plsc.VectorSubcoreMesh(core_axis_name="c", subcore_axis_name="s"),
    compiler_params=pltpu.CompilerParams(use_tc_tiling_on_sc=False),
    scratch_types=[
        pltpu.VMEM((NJ, 16), jnp.int32),
        pltpu.VMEM((NJ, 16), jnp.int32),
        pltpu.VMEM((NJ, 16), jnp.int32),
        pltpu.VMEM((NJ, 16), jnp.int32),
        pltpu.VMEM((SPW * FP,), jnp.int32),
        pltpu.VMEM((PP,), jnp.int32),
        pltpu.VMEM((PP, D), jnp.float32),
        pltpu.VMEM((SPW * FP,), jnp.float32),
        pltpu.VMEM((SPW,), jnp.float32),
        pltpu.VMEM((16,), jnp.float32),
        pltpu.SemaphoreType.DMA,
        pltpu.SemaphoreType.DMA,
    ],
)
def _ffm_sc(x_hbm, tab_hbm, fc_hbm, bias_hbm, sel_a_hbm, sel_b_hbm,
            in_a_hbm, off_hbm, out_hbm,
            sel_a_v, sel_b_v, in_a_v, off_v, x_v, idx_v, rows_v, fc_v,
            out_v, bias_v, sem, sem_fc):
    wid = lax.axis_index("s") * NC + lax.axis_index("c")
    base = wid * SPW

    pltpu.sync_copy(sel_a_hbm, sel_a_v)
    pltpu.sync_copy(sel_b_hbm, sel_b_v)
    pltpu.sync_copy(in_a_hbm, in_a_v)
    pltpu.sync_copy(off_hbm, off_v)
    pltpu.sync_copy(bias_hbm, bias_v)
    pltpu.sync_copy(x_hbm.at[pl.ds(base * FP, SPW * FP)], x_v)
    pltpu.async_copy(fc_hbm.at[x_v], fc_v, sem_fc).wait()

    lane = lax.iota(jnp.int32, 16)

    def sample_body(b, out_vec):
        xa = x_v[pl.ds(b * FP, 16)]
        xb = x_v[pl.ds(b * FP + 16, 16)]
        for j in range(NJ):
            ga = _take16(xa, sel_a_v[j, :])
            gb = _take16(xb, sel_b_v[j, :])
            xg = jnp.where(in_a_v[j, :] > 0, ga, gb)
            idx_v[pl.ds(j * 16, 16)] = xg + off_v[j, :]
        pltpu.async_copy(tab_hbm.at[idx_v], rows_v, sem).wait()

        acc = jnp.zeros((16,), jnp.float32)
        for i in range(F - 1):
            for j in range(i + 1, F):
                acc = acc + rows_v[i * F + j, :] * rows_v[j * F + i, :]

        f0 = fc_v[pl.ds(b * FP, 16)]
        f1 = fc_v[pl.ds(b * FP + 16, 16)]
        acc = acc + f0 + jnp.where(lane < F - 16, f1, 0.0)

        for sh in (8, 4, 2, 1):
            acc = acc + _take16(acc, lane ^ sh)
        out_vec = jnp.where(lane == b % 16, acc, out_vec)
        out_v[pl.ds((b // 16) * 16, 16)] = out_vec
        return out_vec

    lax.fori_loop(0, SPW, sample_body, jnp.zeros((16,), jnp.float32))

    bb = bias_v[:]
    for g in range(SPW // 16):
        zz = out_v[pl.ds(g * 16, 16)] + bb
        out_v[pl.ds(g * 16, 16)] = 1.0 / (1.0 + jnp.exp(-zz))
    pltpu.sync_copy(out_v, out_hbm.at[pl.ds(base, SPW)])


def kernel(x, emb_tables, fc_table, bias):
    sel_a, sel_b, in_a, off = _consts()
    xpad = jnp.pad(x.astype(jnp.int32), ((0, 0), (0, FP - F))).reshape(B * FP)
    tab = emb_tables.reshape(F * V, D)
    fc = fc_table.reshape(V)
    bias16 = jnp.broadcast_to(bias.astype(jnp.float32), (16,))
    return _ffm_sc(xpad, tab, fc, bias16, sel_a, sel_b, in_a, off)


# speedup vs baseline: 9.3668x; 1.0506x over previous
import functools

import jax
import jax.numpy as jnp
import numpy as np
from jax import lax
from jax.experimental import pallas as pl
from jax.experimental.pallas import tpu as pltpu
from jax.experimental.pallas import tpu_sc as plsc

F = 26
FP = 32
V = 100000
D = 16
B = 4096
NC = 2
NS = 16
NW = NC * NS
SPW = B // NW
P = F * (F - 1)
PP = (P + 15) // 16 * 16
NJ = PP // 16


def _pos(s, t):
    return s * (F - 1) + (t - 1 if t > s else t)


def _consts():
    sel = np.zeros(PP, np.int32)
    off = np.zeros(PP, np.int32)
    for s in range(F):
        for t in range(F):
            if t != s:
                sel[_pos(s, t)] = s
                off[_pos(s, t)] = t * V
    sel_a = np.minimum(sel, 15).reshape(NJ, 16)
    sel_b = np.maximum(sel - 16, 0).reshape(NJ, 16)
    in_a = (sel < 16).astype(np.int32).reshape(NJ, 16)
    off = off.reshape(NJ, 16)
    return (jnp.asarray(sel_a), jnp.asarray(sel_b), jnp.asarray(in_a),
            jnp.asarray(off))


def _take16(vec, idx):
    return vec.at[idx].get(mode="promise_in_bounds")


@functools.partial(
    pl.kernel,
    out_type=jax.ShapeDtypeStruct((B,), jnp.float32),
    mesh=plsc.VectorSubcoreMesh(core_axis_name="c", subcore_axis_name="s"),
    compiler_params=pltpu.CompilerParams(use_tc_tiling_on_sc=False),
    scratch_types=[
        pltpu.VMEM((NJ, 16), jnp.int32),
        pltpu.VMEM((NJ, 16), jnp.int32),
        pltpu.VMEM((NJ, 16), jnp.int32),
        pltpu.VMEM((NJ, 16), jnp.int32),
        pltpu.VMEM((SPW * FP,), jnp.int32),
        pltpu.VMEM((PP,), jnp.int32),
        pltpu.VMEM((PP,), jnp.int32),
        pltpu.VMEM((PP, D), jnp.float32),
        pltpu.VMEM((PP, D), jnp.float32),
        pltpu.VMEM((SPW * FP,), jnp.float32),
        pltpu.VMEM((SPW,), jnp.float32),
        pltpu.VMEM((16,), jnp.float32),
        pltpu.SemaphoreType.DMA,
        pltpu.SemaphoreType.DMA,
        pltpu.SemaphoreType.DMA,
    ],
)
def _ffm_sc(x_hbm, tab_hbm, fc_hbm, bias_hbm, sel_a_hbm, sel_b_hbm,
            in_a_hbm, off_hbm, out_hbm,
            sel_a_v, sel_b_v, in_a_v, off_v, x_v, idx0_v, idx1_v,
            rows0_v, rows1_v, fc_v, out_v, bias_v, sem0, sem1, sem_fc):
    wid = lax.axis_index("s") * NC + lax.axis_index("c")
    base = wid * SPW

    pltpu.sync_copy(sel_a_hbm, sel_a_v)
    pltpu.sync_copy(sel_b_hbm, sel_b_v)
    pltpu.sync_copy(in_a_hbm, in_a_v)
    pltpu.sync_copy(off_hbm, off_v)
    pltpu.sync_copy(bias_hbm, bias_v)
    pltpu.sync_copy(x_hbm.at[pl.ds(base * FP, SPW * FP)], x_v)
    pltpu.async_copy(fc_hbm.at[x_v], fc_v, sem_fc).wait()

    lane = lax.iota(jnp.int32, 16)

    def build_fire(b, idx_ref, rows_ref, sem):
        xa = x_v[pl.ds(b * FP, 16)]
        xb = x_v[pl.ds(b * FP + 16, 16)]
        for j in range(NJ):
            ga = _take16(xa, sel_a_v[j, :])
            gb = _take16(xb, sel_b_v[j, :])
            xg = jnp.where(in_a_v[j, :] > 0, ga, gb)
            idx_ref[pl.ds(j * 16, 16)] = xg + off_v[j, :]
        pltpu.async_copy(tab_hbm.at[idx_ref], rows_ref, sem)

    def wait(idx_ref, rows_ref, sem):
        pltpu.make_async_copy(tab_hbm.at[idx_ref], rows_ref, sem).wait()

    def compute(b, rows_ref, out_vec):
        acc = jnp.zeros((16,), jnp.float32)
        for i in range(F - 1):
            for j in range(i + 1, F):
                acc = acc + rows_ref[_pos(i, j), :] * rows_ref[_pos(j, i), :]
        f0 = fc_v[pl.ds(b * FP, 16)]
        f1 = fc_v[pl.ds(b * FP + 16, 16)]
        acc = acc + f0 + jnp.where(lane < F - 16, f1, 0.0)
        for sh in (8, 4, 2, 1):
            acc = acc + _take16(acc, lane ^ sh)
        out_vec = jnp.where(lane == b % 16, acc, out_vec)
        out_v[pl.ds((b // 16) * 16, 16)] = out_vec
        return out_vec

    build_fire(0, idx0_v, rows0_v, sem0)

    def pair_body(g, out_vec):
        b0 = 2 * g
        b1 = b0 + 1
        build_fire(b1, idx1_v, rows1_v, sem1)
        wait(idx0_v, rows0_v, sem0)
        out_vec = compute(b0, rows0_v, out_vec)
        build_fire(jnp.minimum(b1 + 1, SPW - 1), idx0_v, rows0_v, sem0)
        wait(idx1_v, rows1_v, sem1)
        return compute(b1, rows1_v, out_vec)

    lax.fori_loop(0, SPW // 2, pair_body, jnp.zeros((16,), jnp.float32))
    wait(idx0_v, rows0_v, sem0)

    bb = bias_v[:]
    for g in range(SPW // 16):
        zz = out_v[pl.ds(g * 16, 16)] + bb
        out_v[pl.ds(g * 16, 16)] = 1.0 / (1.0 + jnp.exp(-zz))
    pltpu.sync_copy(out_v, out_hbm.at[pl.ds(base, SPW)])


def kernel(x, emb_tables, fc_table, bias):
    sel_a, sel_b, in_a, off = _consts()
    xpad = jnp.pad(x.astype(jnp.int32), ((0, 0), (0, FP - F))).reshape(B * FP)
    tab = emb_tables.reshape(F * V, D)
    fc = fc_table.reshape(V)
    bias16 = jnp.broadcast_to(bias.astype(jnp.float32), (16,))
    return _ffm_sc(xpad, tab, fc, bias16, sel_a, sel_b, in_a, off)
